# Initial kernel scaffold; baseline (speedup 1.0000x reference)
import jax
import jax.numpy as jnp
from jax.experimental import pallas as pl


def kernel(scores):
    raise NotImplementedError("write your pallas kernel here")


# speedup vs baseline: 1.3370x; 1.3370x over previous
import functools

import jax
import jax.numpy as jnp
import numpy as np
from jax.experimental import pallas as pl
from jax.experimental.pallas import tpu as pltpu

_EPS = float(np.finfo(np.float32).tiny)
_K = 16
_TAU = 0.1
_REP = 2


def _body(s_ref, g_ref, o_ref, *, rep, n_iter, k):
    s = s_ref[0]
    sub, lanes = s.shape
    iota = (jax.lax.broadcasted_iota(jnp.int32, (sub, lanes), 0) * lanes
            + jax.lax.broadcasted_iota(jnp.int32, (sub, lanes), 1))
    for r in range(rep):
        x = s + g_ref[r, 0]
        khot = jnp.zeros_like(x)
        for _ in range(n_iter):
            y = x / _TAU
            m = jnp.max(y)
            e = jnp.exp(y - m)
            p = e / jnp.sum(e)
            khot = khot + p
            x = x + jnp.log(jnp.maximum(1.0 - p, _EPS))
        work = khot
        sel = jnp.zeros_like(x, dtype=jnp.bool_)
        for _ in range(k):
            cm = jnp.max(work)
            cand = jnp.where(work == cm, iota, jnp.int32(sub * lanes))
            mi = jnp.min(cand)
            pick = iota == mi
            sel = jnp.logical_or(sel, pick)
            work = jnp.where(pick, -1.0, work)
        o_ref[r, 0] = jnp.where(sel, (1.0 - khot) + khot, 0.0)


def kernel(scores):
    bsz, nmax, ens = scores.shape
    rep = _REP
    k = min(_K, nmax)
    r1 = bsz * ens
    lanes = 128
    sub = nmax // lanes

    s2 = jnp.transpose(scores, (0, 2, 1)).reshape(r1, sub, lanes)
    gkey = jax.random.fold_in(jax.random.key(0), 1)
    g = jax.random.gumbel(gkey, (rep * r1, nmax), dtype=jnp.float32)
    g4 = g.reshape(rep, r1, sub, lanes)

    res = pl.pallas_call(
        functools.partial(_body, rep=rep, n_iter=k, k=k),
        grid=(r1,),
        in_specs=[
            pl.BlockSpec((1, sub, lanes), lambda i: (i, 0, 0)),
            pl.BlockSpec((rep, 1, sub, lanes), lambda i: (0, i, 0, 0)),
        ],
        out_specs=pl.BlockSpec((rep, 1, sub, lanes), lambda i: (0, i, 0, 0)),
        out_shape=jax.ShapeDtypeStruct((rep, r1, sub, lanes), jnp.float32),
        compiler_params=pltpu.CompilerParams(
            dimension_semantics=("arbitrary",),
        ),
    )(s2, g4)

    return res.reshape(rep, bsz, ens, nmax).transpose(0, 1, 3, 2)
